# baseline (speedup 1.0000x reference)
import functools

import jax
import jax.numpy as jnp
from jax import lax
from jax.experimental import pallas as pl
from jax.experimental.pallas import tpu as pltpu
from jax.experimental.pallas import tpu_sc as plsc

NC = 2
NS = 16
LANES = 16


def _make_gather_scale(n, e, d):
    nw = NC * NS
    C = 128
    assert e % (nw * C) == 0
    epw = e // nw
    nch = epw // C
    assert nch % 2 == 0
    ZR = 80
    assert n % ZR == 0
    nzc = n // ZR
    zrounds = -(-nzc // NS)

    mesh = plsc.VectorSubcoreMesh(core_axis_name="c", subcore_axis_name="s")

    @functools.partial(
        pl.kernel,
        out_type=jax.ShapeDtypeStruct((e, d), jnp.float32),
        mesh=mesh,
        scratch_types=[
            pltpu.VMEM_SHARED((n, d), jnp.float32),
            [pltpu.VMEM((C,), jnp.int32)] * 2,
            [pltpu.VMEM((C,), jnp.float32)] * 2,
            [pltpu.VMEM((C, d), jnp.float32)] * 2,
            [pltpu.SemaphoreType.DMA] * 2,
            [pltpu.SemaphoreType.DMA] * 2,
            [pltpu.SemaphoreType.DMA] * 2,
        ],
    )
    def gather_scale(h_hbm, src_hbm, vals_hbm, msg_hbm,
                     hsp, src_v, vals_v, rows_v, sem_sv, sem_g, sem_w):
        c = lax.axis_index("c")
        s = lax.axis_index("s")
        w = s * NC + c
        wbase = w * epw

        for q in range(zrounds):
            t = s + q * NS

            @pl.when(t < nzc)
            def _():
                pltpu.sync_copy(h_hbm.at[pl.ds(t * ZR, ZR)],
                                hsp.at[pl.ds(t * ZR, ZR)])
        plsc.subcore_barrier()

        def issue_sv(k, b):
            base = wbase + k * C
            pltpu.async_copy(src_hbm.at[pl.ds(base, C)], src_v[b], sem_sv[b])
            pltpu.async_copy(vals_hbm.at[pl.ds(base, C)], vals_v[b], sem_sv[b])

        def wait_sv(b):
            pltpu.make_async_copy(src_hbm.at[pl.ds(0, C)], src_v[b],
                                  sem_sv[b]).wait()
            pltpu.make_async_copy(vals_hbm.at[pl.ds(0, C)], vals_v[b],
                                  sem_sv[b]).wait()

        def issue_gather(b):
            pltpu.async_copy(hsp.at[src_v[b]], rows_v[b], sem_g[b])

        def wait_gather(b):
            pltpu.make_async_copy(hsp.at[src_v[b]], rows_v[b],
                                  sem_g[b]).wait()

        def wait_write(b):
            pltpu.make_async_copy(rows_v[b], msg_hbm.at[pl.ds(0, C)],
                                  sem_w[b]).wait()

        issue_sv(0, 0)
        issue_sv(1, 1)
        wait_sv(0)
        issue_gather(0)

        def step(k, cur, nxt):
            @pl.when(k + 1 < nch)
            def _():
                @pl.when(k >= 1)
                def _():
                    wait_write(nxt)
                wait_sv(nxt)
                issue_gather(nxt)
            wait_gather(cur)

            def scale16(g, cc):
                vv = vals_v[cur][pl.ds(g * LANES, LANES)]
                for ii in range(LANES):
                    splat = jnp.broadcast_to(vv[ii], (LANES,))
                    r = g * LANES + ii
                    for j in range(d // LANES):
                        sl = pl.ds(j * LANES, LANES)
                        rows_v[cur][r, sl] = rows_v[cur][r, sl] * splat
                return cc
            lax.fori_loop(0, C // LANES, scale16, 0)

            @pl.when(k + 2 < nch)
            def _():
                issue_sv(k + 2, cur)

            pltpu.async_copy(rows_v[cur], msg_hbm.at[pl.ds(wbase + k * C, C)],
                             sem_w[cur])

        def pair(p, carry):
            step(2 * p, 0, 1)
            step(2 * p + 1, 1, 0)
            return carry
        lax.fori_loop(0, nch // 2, pair, 0)
        wait_write(0)
        wait_write(1)

    return gather_scale


def _make_scatter(n, e, d):
    nw = NC * NS
    C = 64
    NB = 4
    assert e % (nw * C) == 0
    epw = e // nw
    nch = epw // C
    assert nch % NB == 0
    ZR = 40
    assert n % ZR == 0
    nzc = n // ZR
    zrounds = -(-nzc // NS)

    mesh = plsc.VectorSubcoreMesh(core_axis_name="c", subcore_axis_name="s")

    @functools.partial(
        pl.kernel,
        out_type=jax.ShapeDtypeStruct((NC * n, d), jnp.float32),
        mesh=mesh,
        scratch_types=[
            pltpu.VMEM_SHARED((n, d), jnp.float32),
            [pltpu.VMEM((C,), jnp.int32)] * NB,
            [pltpu.VMEM((C, d), jnp.float32)] * NB,
            pltpu.VMEM((ZR, d), jnp.float32),
            [pltpu.SemaphoreType.DMA] * NB,
            [pltpu.SemaphoreType.DMA] * NB,
        ],
    )
    def scatter(msg_hbm, dst_hbm, out_hbm,
                acc, dst_v, rows_v, zbuf, sem_m, sem_s):
        c = lax.axis_index("c")
        s = lax.axis_index("s")
        w = s * NC + c
        wbase = w * epw

        def zrow(i, carry):
            for j in range(d // LANES):
                zbuf[i, pl.ds(j * LANES, LANES)] = jnp.zeros((LANES,), jnp.float32)
            return carry
        lax.fori_loop(0, ZR, zrow, 0)
        for q in range(zrounds):
            t = s + q * NS

            @pl.when(t < nzc)
            def _():
                pltpu.sync_copy(zbuf, acc.at[pl.ds(t * ZR, ZR)])
        plsc.subcore_barrier()

        def issue(k, b):
            base = wbase + k * C
            pltpu.async_copy(dst_hbm.at[pl.ds(base, C)], dst_v[b], sem_m[b])
            pltpu.async_copy(msg_hbm.at[pl.ds(base, C)], rows_v[b], sem_m[b])

        def wait(b):
            pltpu.make_async_copy(dst_hbm.at[pl.ds(0, C)], dst_v[b],
                                  sem_m[b]).wait()
            pltpu.make_async_copy(msg_hbm.at[pl.ds(0, C)], rows_v[b],
                                  sem_m[b]).wait()

        def wait_scatter(b):
            pltpu.make_async_copy(rows_v[b], acc.at[dst_v[b]],
                                  sem_s[b]).wait()

        for b in range(NB - 1):
            issue(b, b)

        def step(k, b, b3):
            wait(b)
            pltpu.async_copy(rows_v[b], acc.at[dst_v[b]], sem_s[b], add=True)

            @pl.when(k + NB - 1 < nch)
            def _():
                @pl.when(k >= 1)
                def _():
                    wait_scatter(b3)
                issue(k + NB - 1, b3)

        def quad(p, carry):
            for j in range(NB):
                step(NB * p + j, j, (j + NB - 1) % NB)
            return carry
        lax.fori_loop(0, nch // NB, quad, 0)
        for b in range(NB):
            wait_scatter(b)

        plsc.subcore_barrier()
        for q in range(zrounds):
            t = s + q * NS

            @pl.when(t < nzc)
            def _():
                pltpu.sync_copy(acc.at[pl.ds(t * ZR, ZR)],
                                out_hbm.at[pl.ds(c * n + t * ZR, ZR)])

    return scatter


def _tc_in(x, w0):
    n, d_in = x.shape
    d_out = w0.shape[1]

    def body(x_ref, w_ref, o_ref):
        xv = x_ref[...]
        mean = jnp.mean(xv, axis=0, keepdims=True)
        var = jnp.mean((xv - mean) ** 2, axis=0, keepdims=True)
        xn = (xv - mean) / jnp.sqrt(var + 1e-5)
        o_ref[...] = jnp.dot(xn, w_ref[...], preferred_element_type=jnp.float32)

    return pl.pallas_call(
        body, out_shape=jax.ShapeDtypeStruct((n, d_out), jnp.float32),
    )(x, w0)


def _tc_mid(p, b, m2, w):
    n2, d = p.shape
    n = n2 // 2
    d_out = w.shape[1]

    def body(p_ref, b_ref, m_ref, w_ref, o_ref):
        pv = p_ref[...]
        h = (pv[:n] + pv[n:] + b_ref[...]) * (m_ref[...].astype(jnp.float32) * 2.0)
        o_ref[...] = jnp.dot(h, w_ref[...], preferred_element_type=jnp.float32)

    return pl.pallas_call(
        body, out_shape=jax.ShapeDtypeStruct((n, d_out), jnp.float32),
    )(p, b.reshape(1, d), m2, w)


def _tc_out(p, b):
    n2, _ = p.shape
    n = n2 // 2
    d = b.shape[0]

    def body(p_ref, b_ref, o_ref):
        pv = p_ref[...]
        z = pv[:n, :d] + pv[n:, :d] + b_ref[...]
        zmax = jnp.max(z, axis=1, keepdims=True)
        zs = z - zmax
        o_ref[...] = zs - jnp.log(jnp.sum(jnp.exp(zs), axis=1, keepdims=True))

    return pl.pallas_call(
        body, out_shape=jax.ShapeDtypeStruct((n, d), jnp.float32),
    )(p, b.reshape(1, d))


def kernel(x, edge_index, adj_vals, W0, b0, W1, b1, W2, b2):
    n, d_in = x.shape
    e = adj_vals.shape[0]
    src = edge_index[0].astype(jnp.int32)
    dst = edge_index[1].astype(jnp.int32)

    ep = -(-e // (NC * NS * 256)) * (NC * NS * 256)
    if ep != e:
        spread = jnp.arange(ep - e, dtype=jnp.int32) % n
        src = jnp.concatenate([src, spread])
        dst = jnp.concatenate([dst, spread])
        adj_vals = jnp.concatenate([adj_vals, jnp.zeros((ep - e,), jnp.float32)])
    e = ep

    d_hid = W0.shape[1]
    d_mid = W1.shape[1]
    d_out = W2.shape[1]

    m0 = jax.random.bernoulli(
        jax.random.fold_in(jax.random.key(42), 0), 0.5, (n, d_hid))
    m1 = jax.random.bernoulli(
        jax.random.fold_in(jax.random.key(42), 1), 0.5, (n, d_mid))

    gs = _make_gather_scale(n, e, d_hid)
    sc = _make_scatter(n, e, d_hid)

    def prop_h(h):
        return sc(gs(h, src, adj_vals), dst)

    W2p = jnp.pad(W2, ((0, 0), (0, d_hid - d_out)))

    h0 = _tc_in(x, W0)
    p0 = prop_h(h0)
    h1 = _tc_mid(p0, b0, m0, W1)
    p1 = prop_h(h1)
    h2 = _tc_mid(p1, b1, m1, W2p)
    p2 = prop_h(h2)
    return _tc_out(p2, b2)


# speedup vs baseline: 1.1295x; 1.1295x over previous
import functools

import jax
import jax.numpy as jnp
from jax import lax
from jax.experimental import pallas as pl
from jax.experimental.pallas import tpu as pltpu
from jax.experimental.pallas import tpu_sc as plsc

NC = 2
NS = 16
LANES = 16


def _make_gather_scale(n, e, d):
    nw = NC * NS
    C = 128
    assert e % (nw * C) == 0
    epw = e // nw
    nch = epw // C
    assert nch % 2 == 0
    ZR = 80
    assert n % ZR == 0
    nzc = n // ZR
    zrounds = -(-nzc // NS)

    mesh = plsc.VectorSubcoreMesh(core_axis_name="c", subcore_axis_name="s")

    @functools.partial(
        pl.kernel,
        out_type=jax.ShapeDtypeStruct((e, d), jnp.float32),
        mesh=mesh,
        scratch_types=[
            pltpu.VMEM_SHARED((n, d), jnp.float32),
            [pltpu.VMEM((C,), jnp.int32)] * 2,
            [pltpu.VMEM((C, d), jnp.float32)] * 2,
            [pltpu.SemaphoreType.DMA] * 2,
            [pltpu.SemaphoreType.DMA] * 2,
            [pltpu.SemaphoreType.DMA] * 2,
        ],
    )
    def gather_scale(h_hbm, src_hbm, msg_hbm,
                     hsp, src_v, rows_v, sem_sv, sem_g, sem_w):
        c = lax.axis_index("c")
        s = lax.axis_index("s")
        w = s * NC + c
        wbase = w * epw

        for q in range(zrounds):
            t = s + q * NS

            @pl.when(t < nzc)
            def _():
                pltpu.sync_copy(h_hbm.at[pl.ds(t * ZR, ZR)],
                                hsp.at[pl.ds(t * ZR, ZR)])
        plsc.subcore_barrier()

        def issue_sv(k, b):
            base = wbase + k * C
            pltpu.async_copy(src_hbm.at[pl.ds(base, C)], src_v[b], sem_sv[b])

        def wait_sv(b):
            pltpu.make_async_copy(src_hbm.at[pl.ds(0, C)], src_v[b],
                                  sem_sv[b]).wait()

        def issue_gather(b):
            pltpu.async_copy(hsp.at[src_v[b]], rows_v[b], sem_g[b])

        def wait_gather(b):
            pltpu.make_async_copy(hsp.at[src_v[b]], rows_v[b],
                                  sem_g[b]).wait()

        def wait_write(b):
            pltpu.make_async_copy(rows_v[b], msg_hbm.at[pl.ds(0, C)],
                                  sem_w[b]).wait()

        issue_sv(0, 0)
        issue_sv(1, 1)
        wait_sv(0)
        issue_gather(0)

        def step(k, cur, nxt):
            @pl.when(k + 1 < nch)
            def _():
                @pl.when(k >= 1)
                def _():
                    wait_write(nxt)
                wait_sv(nxt)
                issue_gather(nxt)
            wait_gather(cur)

            @pl.when(k + 2 < nch)
            def _():
                issue_sv(k + 2, cur)

            pltpu.async_copy(rows_v[cur], msg_hbm.at[pl.ds(wbase + k * C, C)],
                             sem_w[cur])

        def pair(p, carry):
            step(2 * p, 0, 1)
            step(2 * p + 1, 1, 0)
            return carry
        lax.fori_loop(0, nch // 2, pair, 0)
        wait_write(0)
        wait_write(1)

    return gather_scale


def _make_scatter(n, e, d):
    nw = NC * NS
    C = 64
    NB = 4
    assert e % (nw * C) == 0
    epw = e // nw
    nch = epw // C
    assert nch % NB == 0
    ZR = 40
    assert n % ZR == 0
    nzc = n // ZR
    zrounds = -(-nzc // NS)

    mesh = plsc.VectorSubcoreMesh(core_axis_name="c", subcore_axis_name="s")

    @functools.partial(
        pl.kernel,
        out_type=jax.ShapeDtypeStruct((NC * n, d), jnp.float32),
        mesh=mesh,
        scratch_types=[
            pltpu.VMEM_SHARED((n, d), jnp.float32),
            [pltpu.VMEM((C,), jnp.int32)] * NB,
            [pltpu.VMEM((C,), jnp.float32)] * NB,
            [pltpu.VMEM((C, d), jnp.float32)] * NB,
            pltpu.VMEM((ZR, d), jnp.float32),
            [pltpu.SemaphoreType.DMA] * NB,
            [pltpu.SemaphoreType.DMA] * NB,
        ],
    )
    def scatter(msg_hbm, dst_hbm, vals_hbm, out_hbm,
                acc, dst_v, vals_v, rows_v, zbuf, sem_m, sem_s):
        c = lax.axis_index("c")
        s = lax.axis_index("s")
        w = s * NC + c
        wbase = w * epw

        def zrow(i, carry):
            for j in range(d // LANES):
                zbuf[i, pl.ds(j * LANES, LANES)] = jnp.zeros((LANES,), jnp.float32)
            return carry
        lax.fori_loop(0, ZR, zrow, 0)
        for q in range(zrounds):
            t = s + q * NS

            @pl.when(t < nzc)
            def _():
                pltpu.sync_copy(zbuf, acc.at[pl.ds(t * ZR, ZR)])
        plsc.subcore_barrier()

        def issue(k, b):
            base = wbase + k * C
            pltpu.async_copy(dst_hbm.at[pl.ds(base, C)], dst_v[b], sem_m[b])
            pltpu.async_copy(vals_hbm.at[pl.ds(base, C)], vals_v[b], sem_m[b])
            pltpu.async_copy(msg_hbm.at[pl.ds(base, C)], rows_v[b], sem_m[b])

        def wait(b):
            pltpu.make_async_copy(dst_hbm.at[pl.ds(0, C)], dst_v[b],
                                  sem_m[b]).wait()
            pltpu.make_async_copy(vals_hbm.at[pl.ds(0, C)], vals_v[b],
                                  sem_m[b]).wait()
            pltpu.make_async_copy(msg_hbm.at[pl.ds(0, C)], rows_v[b],
                                  sem_m[b]).wait()

        def wait_scatter(b):
            pltpu.make_async_copy(rows_v[b], acc.at[dst_v[b]],
                                  sem_s[b]).wait()

        for b in range(NB - 1):
            issue(b, b)

        def step(k, b, b3):
            wait(b)

            def scale16(g, cc):
                vv = vals_v[b][pl.ds(g * LANES, LANES)]
                for ii in range(LANES):
                    splat = jnp.broadcast_to(vv[ii], (LANES,))
                    r = g * LANES + ii
                    for j in range(d // LANES):
                        sl = pl.ds(j * LANES, LANES)
                        rows_v[b][r, sl] = rows_v[b][r, sl] * splat
                return cc
            lax.fori_loop(0, C // LANES, scale16, 0)

            pltpu.async_copy(rows_v[b], acc.at[dst_v[b]], sem_s[b], add=True)

            @pl.when(k + NB - 1 < nch)
            def _():
                @pl.when(k >= 1)
                def _():
                    wait_scatter(b3)
                issue(k + NB - 1, b3)

        def quad(p, carry):
            for j in range(NB):
                step(NB * p + j, j, (j + NB - 1) % NB)
            return carry
        lax.fori_loop(0, nch // NB, quad, 0)
        for b in range(NB):
            wait_scatter(b)

        plsc.subcore_barrier()
        for q in range(zrounds):
            t = s + q * NS

            @pl.when(t < nzc)
            def _():
                pltpu.sync_copy(acc.at[pl.ds(t * ZR, ZR)],
                                out_hbm.at[pl.ds(c * n + t * ZR, ZR)])

    return scatter


def _tc_in(x, w0):
    n, d_in = x.shape
    d_out = w0.shape[1]

    def body(x_ref, w_ref, o_ref):
        xv = x_ref[...]
        mean = jnp.mean(xv, axis=0, keepdims=True)
        var = jnp.mean((xv - mean) ** 2, axis=0, keepdims=True)
        xn = (xv - mean) / jnp.sqrt(var + 1e-5)
        o_ref[...] = jnp.dot(xn, w_ref[...], preferred_element_type=jnp.float32)

    return pl.pallas_call(
        body, out_shape=jax.ShapeDtypeStruct((n, d_out), jnp.float32),
    )(x, w0)


def _tc_mid(p, b, m2, w):
    n2, d = p.shape
    n = n2 // 2
    d_out = w.shape[1]

    def body(p_ref, b_ref, m_ref, w_ref, o_ref):
        pv = p_ref[...]
        h = (pv[:n] + pv[n:] + b_ref[...]) * (m_ref[...].astype(jnp.float32) * 2.0)
        o_ref[...] = jnp.dot(h, w_ref[...], preferred_element_type=jnp.float32)

    return pl.pallas_call(
        body, out_shape=jax.ShapeDtypeStruct((n, d_out), jnp.float32),
    )(p, b.reshape(1, d), m2, w)


def _tc_out(p, b):
    n2, _ = p.shape
    n = n2 // 2
    d = b.shape[0]

    def body(p_ref, b_ref, o_ref):
        pv = p_ref[...]
        z = pv[:n, :d] + pv[n:, :d] + b_ref[...]
        zmax = jnp.max(z, axis=1, keepdims=True)
        zs = z - zmax
        o_ref[...] = zs - jnp.log(jnp.sum(jnp.exp(zs), axis=1, keepdims=True))

    return pl.pallas_call(
        body, out_shape=jax.ShapeDtypeStruct((n, d), jnp.float32),
    )(p, b.reshape(1, d))


def kernel(x, edge_index, adj_vals, W0, b0, W1, b1, W2, b2):
    n, d_in = x.shape
    e = adj_vals.shape[0]
    src = edge_index[0].astype(jnp.int32)
    dst = edge_index[1].astype(jnp.int32)

    ep = -(-e // (NC * NS * 256)) * (NC * NS * 256)
    if ep != e:
        spread = jnp.arange(ep - e, dtype=jnp.int32) % n
        src = jnp.concatenate([src, spread])
        dst = jnp.concatenate([dst, spread])
        adj_vals = jnp.concatenate([adj_vals, jnp.zeros((ep - e,), jnp.float32)])
    e = ep

    d_hid = W0.shape[1]
    d_mid = W1.shape[1]
    d_out = W2.shape[1]

    m0 = jax.random.bernoulli(
        jax.random.fold_in(jax.random.key(42), 0), 0.5, (n, d_hid))
    m1 = jax.random.bernoulli(
        jax.random.fold_in(jax.random.key(42), 1), 0.5, (n, d_mid))

    gs = _make_gather_scale(n, e, d_hid)
    sc = _make_scatter(n, e, d_hid)

    def prop_h(h):
        return sc(gs(h, src), dst, adj_vals)

    W2p = jnp.pad(W2, ((0, 0), (0, d_hid - d_out)))

    h0 = _tc_in(x, W0)
    p0 = prop_h(h0)
    h1 = _tc_mid(p0, b0, m0, W1)
    p1 = prop_h(h1)
    h2 = _tc_mid(p1, b1, m1, W2p)
    p2 = prop_h(h2)
    return _tc_out(p2, b2)
